# Initial kernel scaffold; baseline (speedup 1.0000x reference)
import jax
import jax.numpy as jnp
from jax.experimental import pallas as pl


def kernel(node_features, edge_index, edge_weight, Wq, bq, Wk, bk, Wv, bv, cbw, Wo, bo):
    raise NotImplementedError("write your pallas kernel here")


# speedup vs baseline: 2.5528x; 2.5528x over previous
import functools

import jax
import jax.numpy as jnp
from jax import lax
from jax.experimental import pallas as pl
from jax.experimental.pallas import tpu as pltpu
from jax.experimental.pallas import tpu_sc as plsc

N = 10000
E = 160000
D = 256
NPAD = 10240
EPAD = 163840
NC = 2
NS = 16
NW = NC * NS
EW = EPAD // NW
C = 160
NCH = EW // C
G = C // 16
RB = 512
NB = NPAD // RB

_DN = (((1,), (1,)), ((), ()))


def _qkv_body(x, wq, wk, wv, bq, bk, bv, q, k, v):
    xv = x[...]
    q[...] = lax.dot_general(xv, wq[...], _DN, preferred_element_type=jnp.float32) + bq[...]
    k[...] = lax.dot_general(xv, wk[...], _DN, preferred_element_type=jnp.float32) + bk[...]
    v[...] = lax.dot_general(xv, wv[...], _DN, preferred_element_type=jnp.float32) + bv[...]


def _out_body(v, sums, wo, bo, out, inv):
    s = sums[0, :] + sums[1, :]
    invd = 1.0 / (s + 1e-9)
    r = s * invd
    upd = v[...] * r[:, None]
    out[...] = lax.dot_general(upd, wo[...], _DN, preferred_element_type=jnp.float32) + bo[...]
    inv[...] = invd


def _edge_body(q_hbm, k_hbm, src_hbm, dst_hbm, ew_hbm, cbw_hbm,
               p_hbm, sums_hbm,
               src_v, dst_v, ew_v, p_v, qrows, krows, cbw_v, zero_v,
               shared_sum, sem):
    c_id = lax.axis_index("c")
    s_id = lax.axis_index("s")
    wid = s_id * NC + c_id
    base_w = wid * EW

    pltpu.sync_copy(cbw_hbm, cbw_v)
    for i in range(32):
        zero_v[pl.ds(i * 16, 16)] = jnp.zeros((16,), jnp.float32)

    @pl.when(s_id == 0)
    def _():
        for i in range(NPAD // 512):
            pltpu.sync_copy(zero_v, shared_sum.at[pl.ds(i * 512, 512)])

    plsc.subcore_barrier()

    lanes = lax.iota(jnp.int32, 16)
    cbw_vec = cbw_v[...]

    @pl.loop(0, NCH)
    def _(ch):
        base_e = base_w + ch * C
        pltpu.sync_copy(src_hbm.at[pl.ds(base_e, C)], src_v)
        pltpu.sync_copy(dst_hbm.at[pl.ds(base_e, C)], dst_v)
        pltpu.sync_copy(ew_hbm.at[pl.ds(base_e, C)], ew_v)
        qd = pltpu.async_copy(q_hbm.at[src_v], qrows, sem)
        kd = pltpu.async_copy(k_hbm.at[dst_v], krows, sem)
        qd.wait()
        kd.wait()
        for g in range(G):
            rows = g * 16 + lanes

            def dot_step(d, acc):
                cols = jnp.full((16,), d, jnp.int32)
                qv = plsc.load_gather(qrows, [rows, cols])
                kv = plsc.load_gather(krows, [rows, cols])
                return acc + qv * kv

            acc = lax.fori_loop(0, D, dot_step, jnp.zeros((16,), jnp.float32),
                                unroll=8)
            ewg = ew_v[pl.ds(g * 16, 16)]
            s = acc * 0.125 + cbw_vec * (ewg + 1.0) * 0.5
            p_v[pl.ds(g * 16, 16)] = jnp.exp(s)
        pltpu.sync_copy(p_v, shared_sum.at[dst_v], add=True)
        pltpu.sync_copy(p_v, p_hbm.at[pl.ds(base_e, C)])

    plsc.subcore_barrier()

    @pl.when(s_id == 0)
    def _():
        pltpu.sync_copy(shared_sum, sums_hbm.at[c_id])


def _wts_body(p_hbm, dst_hbm, inv_hbm, w_hbm, inv_t, p_v, dst_v, w_v):
    c_id = lax.axis_index("c")
    s_id = lax.axis_index("s")
    wid = s_id * NC + c_id
    base = wid * EW
    pltpu.sync_copy(inv_hbm, inv_t)
    pltpu.sync_copy(p_hbm.at[pl.ds(base, EW)], p_v)
    pltpu.sync_copy(dst_hbm.at[pl.ds(base, EW)], dst_v)

    @pl.loop(0, EW // 16, unroll=4)
    def _(g):
        o = g * 16
        idx = dst_v[pl.ds(o, 16)]
        iv = plsc.load_gather(inv_t, [idx])
        w_v[pl.ds(o, 16)] = p_v[pl.ds(o, 16)] * iv

    pltpu.sync_copy(w_v, w_hbm.at[pl.ds(base, EW)])


def kernel(node_features, edge_index, edge_weight, Wq, bq, Wk, bk, Wv, bv, cbw, Wo, bo):
    f32 = jnp.float32
    x = jnp.pad(node_features, ((0, NPAD - N), (0, 0)))
    pad_idx = N + (jnp.arange(EPAD - E, dtype=jnp.int32) % (NPAD - N))
    src = jnp.concatenate([edge_index[0], pad_idx])
    dst = jnp.concatenate([edge_index[1], pad_idx])
    ew = jnp.concatenate([edge_weight.astype(f32), jnp.zeros((EPAD - E,), f32)])
    cbw16 = jnp.broadcast_to(cbw.astype(f32), (16,))
    bq2 = bq.reshape(1, D)
    bk2 = bk.reshape(1, D)
    bv2 = bv.reshape(1, D)
    bo2 = bo.reshape(1, D)

    q, k, v = pl.pallas_call(
        _qkv_body,
        grid=(NB,),
        in_specs=[
            pl.BlockSpec((RB, D), lambda i: (i, 0)),
            pl.BlockSpec((D, D), lambda i: (0, 0)),
            pl.BlockSpec((D, D), lambda i: (0, 0)),
            pl.BlockSpec((D, D), lambda i: (0, 0)),
            pl.BlockSpec((1, D), lambda i: (0, 0)),
            pl.BlockSpec((1, D), lambda i: (0, 0)),
            pl.BlockSpec((1, D), lambda i: (0, 0)),
        ],
        out_specs=[pl.BlockSpec((RB, D), lambda i: (i, 0))] * 3,
        out_shape=[jax.ShapeDtypeStruct((NPAD, D), f32)] * 3,
    )(x, Wq, Wk, Wv, bq2, bk2, bv2)

    mesh = plsc.VectorSubcoreMesh(core_axis_name="c", subcore_axis_name="s")
    p, sums = pl.kernel(
        _edge_body,
        out_type=[
            jax.ShapeDtypeStruct((EPAD,), f32),
            jax.ShapeDtypeStruct((NC, NPAD), f32),
        ],
        mesh=mesh,
        scratch_types=[
            pltpu.VMEM((C,), jnp.int32),
            pltpu.VMEM((C,), jnp.int32),
            pltpu.VMEM((C,), f32),
            pltpu.VMEM((C,), f32),
            pltpu.VMEM((C, D), f32),
            pltpu.VMEM((C, D), f32),
            pltpu.VMEM((16,), f32),
            pltpu.VMEM((512,), f32),
            pltpu.VMEM_SHARED((NPAD,), f32),
            pltpu.SemaphoreType.DMA,
        ],
        compiler_params=pltpu.CompilerParams(
            use_tc_tiling_on_sc=False, needs_layout_passes=False),
    )(q, k, src, dst, ew, cbw16)

    out_pad, inv2d = pl.pallas_call(
        _out_body,
        grid=(NB,),
        in_specs=[
            pl.BlockSpec((RB, D), lambda i: (i, 0)),
            pl.BlockSpec((NC, RB), lambda i: (0, i)),
            pl.BlockSpec((D, D), lambda i: (0, 0)),
            pl.BlockSpec((1, D), lambda i: (0, 0)),
        ],
        out_specs=[
            pl.BlockSpec((RB, D), lambda i: (i, 0)),
            pl.BlockSpec((RB,), lambda i: (i,)),
        ],
        out_shape=[
            jax.ShapeDtypeStruct((NPAD, D), f32),
            jax.ShapeDtypeStruct((NPAD,), f32),
        ],
    )(v, sums, Wo, bo2)

    mesh_b = plsc.VectorSubcoreMesh(core_axis_name="c", subcore_axis_name="s")
    w = pl.kernel(
        _wts_body,
        out_type=jax.ShapeDtypeStruct((EPAD,), f32),
        mesh=mesh_b,
        scratch_types=[
            pltpu.VMEM((NPAD,), f32),
            pltpu.VMEM((EW,), f32),
            pltpu.VMEM((EW,), jnp.int32),
            pltpu.VMEM((EW,), f32),
        ],
        compiler_params=pltpu.CompilerParams(
            use_tc_tiling_on_sc=False, needs_layout_passes=False),
    )(p, dst, inv2d.reshape(-1))

    return out_pad[:N], w[:E]


# speedup vs baseline: 9.3194x; 3.6507x over previous
import functools

import jax
import jax.numpy as jnp
from jax import lax
from jax.experimental import pallas as pl
from jax.experimental.pallas import tpu as pltpu
from jax.experimental.pallas import tpu_sc as plsc

N = 10000
E = 160000
D = 256
NPAD = 10240
EPAD = 163840
NC = 2
NS = 16
NW = NC * NS
EW = EPAD // NW
C = 160
NCH = EW // C
G = C // 16
RB = 512
NB = NPAD // RB

_DN = (((1,), (1,)), ((), ()))


def _qkv_body(x, wq, wk, wv, bq, bk, bv, q, k, v):
    xv = x[...]
    q[...] = lax.dot_general(xv, wq[...], _DN, preferred_element_type=jnp.float32) + bq[...]
    k[...] = lax.dot_general(xv, wk[...], _DN, preferred_element_type=jnp.float32) + bk[...]
    v[...] = lax.dot_general(xv, wv[...], _DN, preferred_element_type=jnp.float32) + bv[...]


def _out_body(v, sums, wo, bo, out, inv):
    s = sums[0, :] + sums[1, :]
    invd = 1.0 / (s + 1e-9)
    r = s * invd
    upd = v[...] * r[:, None]
    out[...] = lax.dot_general(upd, wo[...], _DN, preferred_element_type=jnp.float32) + bo[...]
    inv[...] = invd


def _edge_body(q_hbm, k_hbm, src_hbm, dst_hbm, ew_hbm, cbw_hbm,
               p_hbm, sums_hbm,
               src_v, dst_v, ew_v, p_v, qrows, krows, cbw_v, zero_v, tbuf,
               shared_sum, sem):
    c_id = lax.axis_index("c")
    s_id = lax.axis_index("s")
    wid = s_id * NC + c_id
    base_w = wid * EW

    pltpu.sync_copy(cbw_hbm, cbw_v)
    for i in range(32):
        zero_v[pl.ds(i * 16, 16)] = jnp.zeros((16,), jnp.float32)

    @pl.when(s_id == 0)
    def _():
        for i in range(NPAD // 512):
            pltpu.sync_copy(zero_v, shared_sum.at[pl.ds(i * 512, 512)])

    plsc.subcore_barrier()

    lanes = lax.iota(jnp.int32, 16)
    cbw_vec = cbw_v[...]

    @pl.loop(0, NCH)
    def _(ch):
        base_e = base_w + ch * C
        pltpu.sync_copy(src_hbm.at[pl.ds(base_e, C)], src_v)
        pltpu.sync_copy(dst_hbm.at[pl.ds(base_e, C)], dst_v)
        pltpu.sync_copy(ew_hbm.at[pl.ds(base_e, C)], ew_v)
        qd = pltpu.async_copy(q_hbm.at[src_v], qrows, sem)
        kd = pltpu.async_copy(k_hbm.at[dst_v], krows, sem)
        qd.wait()
        kd.wait()

        @pl.loop(0, G)
        def _(g):
            @pl.loop(0, 16)
            def _(e):
                ge = g * 16 + e
                rows = jnp.full((16,), ge, jnp.int32)

                def jstep(j, acc):
                    cols = j * 16 + lanes
                    qv = plsc.load_gather(qrows, [rows, cols])
                    kv = plsc.load_gather(krows, [rows, cols])
                    return acc + qv * kv

                acc = lax.fori_loop(0, 16, jstep, jnp.zeros((16,), jnp.float32),
                                    unroll=16)
                tbuf[pl.ds(17 * e, 16)] = acc

            s = jnp.zeros((16,), jnp.float32)
            for d in range(16):
                s = s + plsc.load_gather(tbuf, [17 * lanes + d])
            ewg = ew_v[pl.ds(g * 16, 16)]
            s = s * 0.125 + cbw_vec * (ewg + 1.0) * 0.5
            p_v[pl.ds(g * 16, 16)] = jnp.exp(s)
        pltpu.sync_copy(p_v, shared_sum.at[dst_v], add=True)
        pltpu.sync_copy(p_v, p_hbm.at[pl.ds(base_e, C)])

    plsc.subcore_barrier()

    @pl.when(s_id == 0)
    def _():
        pltpu.sync_copy(shared_sum, sums_hbm.at[c_id])


def _wts_body(p_hbm, dst_hbm, inv_hbm, w_hbm, inv_t, p_v, dst_v, w_v):
    c_id = lax.axis_index("c")
    s_id = lax.axis_index("s")
    wid = s_id * NC + c_id
    base = wid * EW
    pltpu.sync_copy(inv_hbm, inv_t)
    pltpu.sync_copy(p_hbm.at[pl.ds(base, EW)], p_v)
    pltpu.sync_copy(dst_hbm.at[pl.ds(base, EW)], dst_v)

    @pl.loop(0, EW // 16, unroll=4)
    def _(g):
        o = g * 16
        idx = dst_v[pl.ds(o, 16)]
        iv = plsc.load_gather(inv_t, [idx])
        w_v[pl.ds(o, 16)] = p_v[pl.ds(o, 16)] * iv

    pltpu.sync_copy(w_v, w_hbm.at[pl.ds(base, EW)])


def kernel(node_features, edge_index, edge_weight, Wq, bq, Wk, bk, Wv, bv, cbw, Wo, bo):
    f32 = jnp.float32
    x = jnp.pad(node_features, ((0, NPAD - N), (0, 0)))
    pad_idx = N + (jnp.arange(EPAD - E, dtype=jnp.int32) % (NPAD - N))
    src = jnp.concatenate([edge_index[0], pad_idx])
    dst = jnp.concatenate([edge_index[1], pad_idx])
    ew = jnp.concatenate([edge_weight.astype(f32), jnp.zeros((EPAD - E,), f32)])
    cbw16 = jnp.broadcast_to(cbw.astype(f32), (16,))
    bq2 = bq.reshape(1, D)
    bk2 = bk.reshape(1, D)
    bv2 = bv.reshape(1, D)
    bo2 = bo.reshape(1, D)

    q, k, v = pl.pallas_call(
        _qkv_body,
        grid=(NB,),
        in_specs=[
            pl.BlockSpec((RB, D), lambda i: (i, 0)),
            pl.BlockSpec((D, D), lambda i: (0, 0)),
            pl.BlockSpec((D, D), lambda i: (0, 0)),
            pl.BlockSpec((D, D), lambda i: (0, 0)),
            pl.BlockSpec((1, D), lambda i: (0, 0)),
            pl.BlockSpec((1, D), lambda i: (0, 0)),
            pl.BlockSpec((1, D), lambda i: (0, 0)),
        ],
        out_specs=[pl.BlockSpec((RB, D), lambda i: (i, 0))] * 3,
        out_shape=[jax.ShapeDtypeStruct((NPAD, D), f32)] * 3,
    )(x, Wq, Wk, Wv, bq2, bk2, bv2)

    mesh = plsc.VectorSubcoreMesh(core_axis_name="c", subcore_axis_name="s")
    p, sums = pl.kernel(
        _edge_body,
        out_type=[
            jax.ShapeDtypeStruct((EPAD,), f32),
            jax.ShapeDtypeStruct((NC, NPAD), f32),
        ],
        mesh=mesh,
        scratch_types=[
            pltpu.VMEM((C,), jnp.int32),
            pltpu.VMEM((C,), jnp.int32),
            pltpu.VMEM((C,), f32),
            pltpu.VMEM((C,), f32),
            pltpu.VMEM((C, D), f32),
            pltpu.VMEM((C, D), f32),
            pltpu.VMEM((16,), f32),
            pltpu.VMEM((512,), f32),
            pltpu.VMEM((17 * 16,), f32),
            pltpu.VMEM_SHARED((NPAD,), f32),
            pltpu.SemaphoreType.DMA,
        ],
        compiler_params=pltpu.CompilerParams(
            use_tc_tiling_on_sc=False, needs_layout_passes=False),
    )(q, k, src, dst, ew, cbw16)

    out_pad, inv2d = pl.pallas_call(
        _out_body,
        grid=(NB,),
        in_specs=[
            pl.BlockSpec((RB, D), lambda i: (i, 0)),
            pl.BlockSpec((NC, RB), lambda i: (0, i)),
            pl.BlockSpec((D, D), lambda i: (0, 0)),
            pl.BlockSpec((1, D), lambda i: (0, 0)),
        ],
        out_specs=[
            pl.BlockSpec((RB, D), lambda i: (i, 0)),
            pl.BlockSpec((RB,), lambda i: (i,)),
        ],
        out_shape=[
            jax.ShapeDtypeStruct((NPAD, D), f32),
            jax.ShapeDtypeStruct((NPAD,), f32),
        ],
    )(v, sums, Wo, bo2)

    mesh_b = plsc.VectorSubcoreMesh(core_axis_name="c", subcore_axis_name="s")
    w = pl.kernel(
        _wts_body,
        out_type=jax.ShapeDtypeStruct((EPAD,), f32),
        mesh=mesh_b,
        scratch_types=[
            pltpu.VMEM((NPAD,), f32),
            pltpu.VMEM((EW,), f32),
            pltpu.VMEM((EW,), jnp.int32),
            pltpu.VMEM((EW,), f32),
        ],
        compiler_params=pltpu.CompilerParams(
            use_tc_tiling_on_sc=False, needs_layout_passes=False),
    )(p, dst, inv2d.reshape(-1))

    return out_pad[:N], w[:E]


# speedup vs baseline: 14.5222x; 1.5583x over previous
import functools

import jax
import jax.numpy as jnp
from jax import lax
from jax.experimental import pallas as pl
from jax.experimental.pallas import tpu as pltpu
from jax.experimental.pallas import tpu_sc as plsc

N = 10000
E = 160000
D = 256
NPAD = 10240
EPAD = 163840
NC = 2
NS = 16
NW = NC * NS
EW = EPAD // NW
C = 80
NCH = EW // C
G = C // 16
RB = 512
NB = NPAD // RB

_DN = (((1,), (1,)), ((), ()))


def _qkv_body(x, wq, wk, wv, bq, bk, bv, q, k, v):
    xv = x[...]
    q[...] = lax.dot_general(xv, wq[...], _DN, preferred_element_type=jnp.float32) + bq[...]
    k[...] = lax.dot_general(xv, wk[...], _DN, preferred_element_type=jnp.float32) + bk[...]
    v[...] = lax.dot_general(xv, wv[...], _DN, preferred_element_type=jnp.float32) + bv[...]


def _out_body(v, sums, wo, bo, out, inv):
    s = sums[0, :] + sums[1, :]
    invd = 1.0 / (s + 1e-9)
    r = s * invd
    upd = v[...] * r[:, None]
    out[...] = lax.dot_general(upd, wo[...], _DN, preferred_element_type=jnp.float32) + bo[...]
    inv[...] = invd


def _edge_body(q_hbm, k_hbm, src_hbm, dst_hbm, ew_hbm, cbw_hbm,
               p_hbm, sums_hbm,
               src_all, dst_all, ew_all, dst_v, p_v, qrows, krows, cbw_v,
               zero_v, tbuf, shared_sum, sem):
    c_id = lax.axis_index("c")
    s_id = lax.axis_index("s")
    wid = s_id * NC + c_id
    base_w = wid * EW

    pltpu.sync_copy(cbw_hbm, cbw_v)
    for i in range(32):
        zero_v[pl.ds(i * 16, 16)] = jnp.zeros((16,), jnp.float32)

    @pl.when(s_id == 0)
    def _():
        for i in range(NPAD // 512):
            pltpu.sync_copy(zero_v, shared_sum.at[pl.ds(i * 512, 512)])

    plsc.subcore_barrier()

    lanes = lax.iota(jnp.int32, 16)
    cbw_vec = cbw_v[...]

    pltpu.sync_copy(src_hbm.at[pl.ds(base_w, EW)], src_all)
    pltpu.sync_copy(dst_hbm.at[pl.ds(base_w, EW)], dst_all)
    pltpu.sync_copy(ew_hbm.at[pl.ds(base_w, EW)], ew_all)

    def start_chunk(b, ch):
        pltpu.async_copy(dst_hbm.at[pl.ds(base_w + ch * C, C)], dst_v[b], sem[b])
        pltpu.async_copy(q_hbm.at[src_all.at[pl.ds(ch * C, C)]], qrows[b], sem[b])
        pltpu.async_copy(k_hbm.at[dst_all.at[pl.ds(ch * C, C)]], krows[b], sem[b])

    def wait_chunk(b, ch):
        pltpu.make_async_copy(dst_hbm.at[pl.ds(base_w + ch * C, C)], dst_v[b], sem[b]).wait()
        pltpu.make_async_copy(q_hbm.at[src_all.at[pl.ds(ch * C, C)]], qrows[b], sem[b]).wait()
        pltpu.make_async_copy(k_hbm.at[dst_all.at[pl.ds(ch * C, C)]], krows[b], sem[b]).wait()

    def compute_chunk(b, ch):
        base_e = base_w + ch * C

        @pl.loop(0, G)
        def _(g):
            @pl.loop(0, 16)
            def _(e):
                ge = g * 16 + e
                rows = jnp.full((16,), ge, jnp.int32)

                def jstep(j, acc):
                    cols = j * 16 + lanes
                    qv = plsc.load_gather(qrows[b], [rows, cols])
                    kv = plsc.load_gather(krows[b], [rows, cols])
                    return acc + qv * kv

                acc = lax.fori_loop(0, 16, jstep, jnp.zeros((16,), jnp.float32),
                                    unroll=16)
                tbuf[pl.ds(17 * e, 16)] = acc

            s = jnp.zeros((16,), jnp.float32)
            for d in range(16):
                s = s + plsc.load_gather(tbuf, [17 * lanes + d])
            ewg = ew_all[pl.ds(ch * C + g * 16, 16)]
            s = s * 0.125 + cbw_vec * (ewg + 1.0) * 0.5
            p_v[pl.ds(g * 16, 16)] = jnp.exp(s)
        pltpu.sync_copy(p_v, shared_sum.at[dst_v[b]], add=True)
        pltpu.sync_copy(p_v, p_hbm.at[pl.ds(base_e, C)])

    start_chunk(0, 0)

    @pl.loop(0, NCH, step=2)
    def _(cc):
        for b in range(2):
            ch = cc + b

            @pl.when(ch + 1 < NCH)
            def _():
                start_chunk(1 - b, ch + 1)

            wait_chunk(b, ch)
            compute_chunk(b, ch)

    plsc.subcore_barrier()

    @pl.when(s_id == 0)
    def _():
        pltpu.sync_copy(shared_sum, sums_hbm.at[c_id])


def _wts_body(p_hbm, dst_hbm, inv_hbm, w_hbm, inv_t, p_v, dst_v, w_v):
    c_id = lax.axis_index("c")
    s_id = lax.axis_index("s")
    wid = s_id * NC + c_id
    base = wid * EW
    pltpu.sync_copy(inv_hbm, inv_t)
    pltpu.sync_copy(p_hbm.at[pl.ds(base, EW)], p_v)
    pltpu.sync_copy(dst_hbm.at[pl.ds(base, EW)], dst_v)

    @pl.loop(0, EW // 16, unroll=4)
    def _(g):
        o = g * 16
        idx = dst_v[pl.ds(o, 16)]
        iv = plsc.load_gather(inv_t, [idx])
        w_v[pl.ds(o, 16)] = p_v[pl.ds(o, 16)] * iv

    pltpu.sync_copy(w_v, w_hbm.at[pl.ds(base, EW)])


def kernel(node_features, edge_index, edge_weight, Wq, bq, Wk, bk, Wv, bv, cbw, Wo, bo):
    f32 = jnp.float32
    x = jnp.pad(node_features, ((0, NPAD - N), (0, 0)))
    pad_idx = N + (jnp.arange(EPAD - E, dtype=jnp.int32) % (NPAD - N))
    src = jnp.concatenate([edge_index[0], pad_idx])
    dst = jnp.concatenate([edge_index[1], pad_idx])
    ew = jnp.concatenate([edge_weight.astype(f32), jnp.zeros((EPAD - E,), f32)])
    cbw16 = jnp.broadcast_to(cbw.astype(f32), (16,))
    bq2 = bq.reshape(1, D)
    bk2 = bk.reshape(1, D)
    bv2 = bv.reshape(1, D)
    bo2 = bo.reshape(1, D)

    q, k, v = pl.pallas_call(
        _qkv_body,
        grid=(NB,),
        in_specs=[
            pl.BlockSpec((RB, D), lambda i: (i, 0)),
            pl.BlockSpec((D, D), lambda i: (0, 0)),
            pl.BlockSpec((D, D), lambda i: (0, 0)),
            pl.BlockSpec((D, D), lambda i: (0, 0)),
            pl.BlockSpec((1, D), lambda i: (0, 0)),
            pl.BlockSpec((1, D), lambda i: (0, 0)),
            pl.BlockSpec((1, D), lambda i: (0, 0)),
        ],
        out_specs=[pl.BlockSpec((RB, D), lambda i: (i, 0))] * 3,
        out_shape=[jax.ShapeDtypeStruct((NPAD, D), f32)] * 3,
    )(x, Wq, Wk, Wv, bq2, bk2, bv2)

    mesh = plsc.VectorSubcoreMesh(core_axis_name="c", subcore_axis_name="s")
    p, sums = pl.kernel(
        _edge_body,
        out_type=[
            jax.ShapeDtypeStruct((EPAD,), f32),
            jax.ShapeDtypeStruct((NC, NPAD), f32),
        ],
        mesh=mesh,
        scratch_types=[
            pltpu.VMEM((EW,), jnp.int32),
            pltpu.VMEM((EW,), jnp.int32),
            pltpu.VMEM((EW,), f32),
            [pltpu.VMEM((C,), jnp.int32)] * 2,
            pltpu.VMEM((C,), f32),
            [pltpu.VMEM((C, D), f32)] * 2,
            [pltpu.VMEM((C, D), f32)] * 2,
            pltpu.VMEM((16,), f32),
            pltpu.VMEM((512,), f32),
            pltpu.VMEM((17 * 16,), f32),
            pltpu.VMEM_SHARED((NPAD,), f32),
            [pltpu.SemaphoreType.DMA] * 2,
        ],
        compiler_params=pltpu.CompilerParams(
            use_tc_tiling_on_sc=False, needs_layout_passes=False),
    )(q, k, src, dst, ew, cbw16)

    out_pad, inv2d = pl.pallas_call(
        _out_body,
        grid=(NB,),
        in_specs=[
            pl.BlockSpec((RB, D), lambda i: (i, 0)),
            pl.BlockSpec((NC, RB), lambda i: (0, i)),
            pl.BlockSpec((D, D), lambda i: (0, 0)),
            pl.BlockSpec((1, D), lambda i: (0, 0)),
        ],
        out_specs=[
            pl.BlockSpec((RB, D), lambda i: (i, 0)),
            pl.BlockSpec((RB,), lambda i: (i,)),
        ],
        out_shape=[
            jax.ShapeDtypeStruct((NPAD, D), f32),
            jax.ShapeDtypeStruct((NPAD,), f32),
        ],
    )(v, sums, Wo, bo2)

    mesh_b = plsc.VectorSubcoreMesh(core_axis_name="c", subcore_axis_name="s")
    w = pl.kernel(
        _wts_body,
        out_type=jax.ShapeDtypeStruct((EPAD,), f32),
        mesh=mesh_b,
        scratch_types=[
            pltpu.VMEM((NPAD,), f32),
            pltpu.VMEM((EW,), f32),
            pltpu.VMEM((EW,), jnp.int32),
            pltpu.VMEM((EW,), f32),
        ],
        compiler_params=pltpu.CompilerParams(
            use_tc_tiling_on_sc=False, needs_layout_passes=False),
    )(p, dst, inv2d.reshape(-1))

    return out_pad[:N], w[:E]


# speedup vs baseline: 15.4407x; 1.0632x over previous
import functools

import jax
import jax.numpy as jnp
from jax import lax
from jax.experimental import pallas as pl
from jax.experimental.pallas import tpu as pltpu
from jax.experimental.pallas import tpu_sc as plsc

N = 10000
E = 160000
D = 256
EPAD = 163840
NC = 2
NS = 16
NW = NC * NS
EW = EPAD // NW
C = 80
NCH = EW // C
G = C // 16
RB = 1000
NB = N // RB

_DN = (((1,), (1,)), ((), ()))


def _qk_body(x, wq, wk, bq, bk, q, k):
    xv = x[...]
    q[...] = lax.dot_general(xv, wq[...], _DN, preferred_element_type=jnp.float32) + bq[...]
    k[...] = lax.dot_general(xv, wk[...], _DN, preferred_element_type=jnp.float32) + bk[...]


def _out_body(x, r, wv, bv, wo, bo, out):
    v = lax.dot_general(x[...], wv[...], _DN, preferred_element_type=jnp.float32) + bv[...]
    out[...] = lax.dot_general(v * r[...], wo[...], _DN,
                               preferred_element_type=jnp.float32) + bo[...]


def _edge_body(q_hbm, k_hbm, src_hbm, dst_hbm, ew_hbm, cbw_hbm,
               p_hbm, sums_hbm,
               src_all, dst_all, ew_all, dst_v, p_v, qrows, krows, cbw_v,
               zero_v, tbuf, shared_sum, sem):
    c_id = lax.axis_index("c")
    s_id = lax.axis_index("s")
    wid = s_id * NC + c_id
    base_w = wid * EW

    pltpu.sync_copy(cbw_hbm, cbw_v)

    @pl.loop(0, 2000 // 16)
    def _(i):
        zero_v[pl.ds(i * 16, 16)] = jnp.zeros((16,), jnp.float32)

    @pl.when(s_id == 0)
    def _():
        for i in range(N // 2000):
            pltpu.sync_copy(zero_v, shared_sum.at[pl.ds(i * 2000, 2000)])

    plsc.subcore_barrier()

    lanes = lax.iota(jnp.int32, 16)
    cbw_vec = cbw_v[...]

    pltpu.sync_copy(src_hbm.at[pl.ds(base_w, EW)], src_all)
    pltpu.sync_copy(dst_hbm.at[pl.ds(base_w, EW)], dst_all)
    pltpu.sync_copy(ew_hbm.at[pl.ds(base_w, EW)], ew_all)

    def start_chunk(b, ch):
        pltpu.async_copy(dst_hbm.at[pl.ds(base_w + ch * C, C)], dst_v[b], sem[b])
        pltpu.async_copy(q_hbm.at[src_all.at[pl.ds(ch * C, C)]], qrows[b], sem[b])
        pltpu.async_copy(k_hbm.at[dst_all.at[pl.ds(ch * C, C)]], krows[b], sem[b])

    def wait_chunk(b, ch):
        pltpu.make_async_copy(dst_hbm.at[pl.ds(base_w + ch * C, C)], dst_v[b], sem[b]).wait()
        pltpu.make_async_copy(q_hbm.at[src_all.at[pl.ds(ch * C, C)]], qrows[b], sem[b]).wait()
        pltpu.make_async_copy(k_hbm.at[dst_all.at[pl.ds(ch * C, C)]], krows[b], sem[b]).wait()

    def compute_chunk(b, ch):
        base_e = base_w + ch * C

        @pl.loop(0, G)
        def _(g):
            @pl.loop(0, 16)
            def _(e):
                ge = g * 16 + e
                rows = jnp.full((16,), ge, jnp.int32)

                def jstep(j, acc):
                    cols = j * 16 + lanes
                    qv = plsc.load_gather(qrows[b], [rows, cols])
                    kv = plsc.load_gather(krows[b], [rows, cols])
                    return acc + qv * kv

                acc = lax.fori_loop(0, 16, jstep, jnp.zeros((16,), jnp.float32),
                                    unroll=16)
                tbuf[pl.ds(17 * e, 16)] = acc

            s = jnp.zeros((16,), jnp.float32)
            for d in range(16):
                s = s + plsc.load_gather(tbuf, [17 * lanes + d])
            ewg = ew_all[pl.ds(ch * C + g * 16, 16)]
            s = s * 0.125 + cbw_vec * (ewg + 1.0) * 0.5
            ge = base_e + g * 16 + lanes
            p_v[pl.ds(g * 16, 16)] = jnp.where(ge < E, jnp.exp(s), 0.0)
        pltpu.sync_copy(p_v, shared_sum.at[dst_v[b]], add=True)
        pltpu.sync_copy(p_v, p_hbm.at[pl.ds(base_e, C)])

    start_chunk(0, 0)

    @pl.loop(0, NCH, step=2)
    def _(cc):
        for b in range(2):
            ch = cc + b

            @pl.when(ch + 1 < NCH)
            def _():
                start_chunk(1 - b, ch + 1)

            wait_chunk(b, ch)
            compute_chunk(b, ch)

    plsc.subcore_barrier()

    @pl.when(s_id == 0)
    def _():
        pltpu.sync_copy(shared_sum, sums_hbm.at[c_id])


def _wts_body(p_hbm, dst_hbm, sums_hbm, w_hbm, r_hbm,
              s_all, inv_t, r_t, p_v, dst_v, w_v):
    c_id = lax.axis_index("c")
    s_id = lax.axis_index("s")
    wid = s_id * NC + c_id
    base = wid * EW
    pltpu.sync_copy(sums_hbm, s_all)
    pltpu.sync_copy(p_hbm.at[pl.ds(base, EW)], p_v)
    pltpu.sync_copy(dst_hbm.at[pl.ds(base, EW)], dst_v)

    @pl.loop(0, N // 16)
    def _(i):
        o = i * 16
        tot = s_all[pl.ds(o, 16)] + s_all[pl.ds(N + o, 16)]
        iv = 1.0 / (tot + 1e-9)
        inv_t[pl.ds(o, 16)] = iv
        r_t[pl.ds(o, 16)] = tot * iv

    @pl.when(wid == 0)
    def _():
        pltpu.sync_copy(r_t, r_hbm)

    @pl.loop(0, EW // 16, unroll=4)
    def _(g):
        o = g * 16
        idx = dst_v[pl.ds(o, 16)]
        iv = plsc.load_gather(inv_t, [idx])
        w_v[pl.ds(o, 16)] = p_v[pl.ds(o, 16)] * iv

    pltpu.sync_copy(w_v, w_hbm.at[pl.ds(base, EW)])


def kernel(node_features, edge_index, edge_weight, Wq, bq, Wk, bk, Wv, bv, cbw, Wo, bo):
    f32 = jnp.float32
    pad_idx = jnp.arange(EPAD - E, dtype=jnp.int32) % N
    src = jnp.concatenate([edge_index[0], pad_idx])
    dst = jnp.concatenate([edge_index[1], pad_idx])
    ew = jnp.concatenate([edge_weight.astype(f32), jnp.zeros((EPAD - E,), f32)])
    cbw16 = jnp.broadcast_to(cbw.astype(f32), (16,))
    bq2 = bq.reshape(1, D)
    bk2 = bk.reshape(1, D)
    bv2 = bv.reshape(1, D)
    bo2 = bo.reshape(1, D)

    q, k = pl.pallas_call(
        _qk_body,
        grid=(NB,),
        in_specs=[
            pl.BlockSpec((RB, D), lambda i: (i, 0)),
            pl.BlockSpec((D, D), lambda i: (0, 0)),
            pl.BlockSpec((D, D), lambda i: (0, 0)),
            pl.BlockSpec((1, D), lambda i: (0, 0)),
            pl.BlockSpec((1, D), lambda i: (0, 0)),
        ],
        out_specs=[pl.BlockSpec((RB, D), lambda i: (i, 0))] * 2,
        out_shape=[jax.ShapeDtypeStruct((N, D), f32)] * 2,
    )(node_features, Wq, Wk, bq2, bk2)

    mesh = plsc.VectorSubcoreMesh(core_axis_name="c", subcore_axis_name="s")
    p, sums = pl.kernel(
        _edge_body,
        out_type=[
            jax.ShapeDtypeStruct((EPAD,), f32),
            jax.ShapeDtypeStruct((NC, N), f32),
        ],
        mesh=mesh,
        scratch_types=[
            pltpu.VMEM((EW,), jnp.int32),
            pltpu.VMEM((EW,), jnp.int32),
            pltpu.VMEM((EW,), f32),
            [pltpu.VMEM((C,), jnp.int32)] * 2,
            pltpu.VMEM((C,), f32),
            [pltpu.VMEM((C, D), f32)] * 2,
            [pltpu.VMEM((C, D), f32)] * 2,
            pltpu.VMEM((16,), f32),
            pltpu.VMEM((2000,), f32),
            pltpu.VMEM((17 * 16,), f32),
            pltpu.VMEM_SHARED((N,), f32),
            [pltpu.SemaphoreType.DMA] * 2,
        ],
        compiler_params=pltpu.CompilerParams(
            use_tc_tiling_on_sc=False, needs_layout_passes=False),
    )(q, k, src, dst, ew, cbw16)

    mesh_b = plsc.VectorSubcoreMesh(core_axis_name="c", subcore_axis_name="s")
    w, r = pl.kernel(
        _wts_body,
        out_type=[
            jax.ShapeDtypeStruct((EPAD,), f32),
            jax.ShapeDtypeStruct((N,), f32),
        ],
        mesh=mesh_b,
        scratch_types=[
            pltpu.VMEM((NC * N,), f32),
            pltpu.VMEM((N,), f32),
            pltpu.VMEM((N,), f32),
            pltpu.VMEM((EW,), f32),
            pltpu.VMEM((EW,), jnp.int32),
            pltpu.VMEM((EW,), f32),
        ],
        compiler_params=pltpu.CompilerParams(
            use_tc_tiling_on_sc=False, needs_layout_passes=False),
    )(p, dst, sums.reshape(-1))

    out = pl.pallas_call(
        _out_body,
        grid=(NB,),
        in_specs=[
            pl.BlockSpec((RB, D), lambda i: (i, 0)),
            pl.BlockSpec((RB, 1), lambda i: (i, 0)),
            pl.BlockSpec((D, D), lambda i: (0, 0)),
            pl.BlockSpec((1, D), lambda i: (0, 0)),
            pl.BlockSpec((D, D), lambda i: (0, 0)),
            pl.BlockSpec((1, D), lambda i: (0, 0)),
        ],
        out_specs=pl.BlockSpec((RB, D), lambda i: (i, 0)),
        out_shape=jax.ShapeDtypeStruct((N, D), f32),
    )(node_features, r.reshape(N, 1), Wv, bv2, Wo, bo2)

    return out, w[:E]


# speedup vs baseline: 16.7655x; 1.0858x over previous
import functools

import jax
import jax.numpy as jnp
from jax import lax
from jax.experimental import pallas as pl
from jax.experimental.pallas import tpu as pltpu
from jax.experimental.pallas import tpu_sc as plsc

N = 10000
E = 160000
D = 256
DP = 128
EPAD = 163840
NC = 2
NS = 16
NW = NC * NS
EW = EPAD // NW
C = 80
NCH = EW // C
G = C // 16
RB = 1000
NB = N // RB

_DN = (((1,), (1,)), ((), ()))


def _pack_bf16(a):
    a16 = lax.bitcast_convert_type(a.astype(jnp.bfloat16), jnp.uint16)
    lo = a16[:, :128].astype(jnp.uint32)
    hi = a16[:, 128:].astype(jnp.uint32)
    return lax.bitcast_convert_type((hi << 16) | lo, jnp.float32)


def _qk_body(x, wq, wk, bq, bk, q, k):
    xv = x[...]
    q[...] = _pack_bf16(
        lax.dot_general(xv, wq[...], _DN, preferred_element_type=jnp.float32) + bq[...])
    k[...] = _pack_bf16(
        lax.dot_general(xv, wk[...], _DN, preferred_element_type=jnp.float32) + bk[...])


def _out_body(x, r, wv, bv, wo, bo, out):
    v = lax.dot_general(x[...], wv[...], _DN, preferred_element_type=jnp.float32) + bv[...]
    out[...] = lax.dot_general(v * r[...], wo[...], _DN,
                               preferred_element_type=jnp.float32) + bo[...]


def _edge_body(q_hbm, k_hbm, src_hbm, dst_hbm, ew_hbm, cbw_hbm,
               p_hbm, sums_hbm,
               src_all, dst_all, ew_all, dst_v, p_v, qrows, krows, cbw_v,
               zero_v, tbuf, shared_sum, sem):
    c_id = lax.axis_index("c")
    s_id = lax.axis_index("s")
    wid = s_id * NC + c_id
    base_w = wid * EW

    pltpu.sync_copy(cbw_hbm, cbw_v)

    @pl.loop(0, 2000 // 16)
    def _(i):
        zero_v[pl.ds(i * 16, 16)] = jnp.zeros((16,), jnp.float32)

    @pl.when(s_id == 0)
    def _():
        for i in range(N // 2000):
            pltpu.sync_copy(zero_v, shared_sum.at[pl.ds(i * 2000, 2000)])

    plsc.subcore_barrier()

    lanes = lax.iota(jnp.int32, 16)
    cbw_vec = cbw_v[...]

    pltpu.sync_copy(src_hbm.at[pl.ds(base_w, EW)], src_all)
    pltpu.sync_copy(dst_hbm.at[pl.ds(base_w, EW)], dst_all)
    pltpu.sync_copy(ew_hbm.at[pl.ds(base_w, EW)], ew_all)

    def start_chunk(b, ch):
        pltpu.async_copy(dst_hbm.at[pl.ds(base_w + ch * C, C)], dst_v[b], sem[b])
        pltpu.async_copy(q_hbm.at[src_all.at[pl.ds(ch * C, C)]], qrows[b], sem[b])
        pltpu.async_copy(k_hbm.at[dst_all.at[pl.ds(ch * C, C)]], krows[b], sem[b])

    def wait_chunk(b, ch):
        pltpu.make_async_copy(dst_hbm.at[pl.ds(base_w + ch * C, C)], dst_v[b], sem[b]).wait()
        pltpu.make_async_copy(q_hbm.at[src_all.at[pl.ds(ch * C, C)]], qrows[b], sem[b]).wait()
        pltpu.make_async_copy(k_hbm.at[dst_all.at[pl.ds(ch * C, C)]], krows[b], sem[b]).wait()

    def compute_chunk(b, ch):
        base_e = base_w + ch * C

        @pl.loop(0, G)
        def _(g):
            @pl.loop(0, 16)
            def _(e):
                ge = g * 16 + e
                rows = jnp.full((16,), ge, jnp.int32)

                def jstep(j, acc):
                    cols = j * 16 + lanes
                    qw = plsc.load_gather(qrows[b], [rows, cols])
                    kw = plsc.load_gather(krows[b], [rows, cols])
                    pr = plsc.bitcast(qw, jnp.bfloat16) * plsc.bitcast(kw, jnp.bfloat16)
                    pa, pb = plsc.unpack(pr, format=plsc.PackFormat.INTERLEAVED)
                    return acc + pa + pb

                acc = lax.fori_loop(0, DP // 16, jstep,
                                    jnp.zeros((16,), jnp.float32), unroll=8)
                tbuf[pl.ds(17 * e, 16)] = acc

            s = jnp.zeros((16,), jnp.float32)
            for d in range(16):
                s = s + plsc.load_gather(tbuf, [17 * lanes + d])
            ewg = ew_all[pl.ds(ch * C + g * 16, 16)]
            s = s * 0.125 + cbw_vec * (ewg + 1.0) * 0.5
            ge = base_e + g * 16 + lanes
            p_v[pl.ds(g * 16, 16)] = jnp.where(ge < E, jnp.exp(s), 0.0)
        pltpu.sync_copy(p_v, shared_sum.at[dst_v[b]], add=True)
        pltpu.sync_copy(p_v, p_hbm.at[pl.ds(base_e, C)])

    start_chunk(0, 0)

    @pl.loop(0, NCH, step=2)
    def _(cc):
        for b in range(2):
            ch = cc + b

            @pl.when(ch + 1 < NCH)
            def _():
                start_chunk(1 - b, ch + 1)

            wait_chunk(b, ch)
            compute_chunk(b, ch)

    plsc.subcore_barrier()

    @pl.when(s_id == 0)
    def _():
        pltpu.sync_copy(shared_sum, sums_hbm.at[c_id])


def _wts_body(p_hbm, dst_hbm, sums_hbm, w_hbm, r_hbm,
              s_all, inv_t, r_t, p_v, dst_v, w_v):
    c_id = lax.axis_index("c")
    s_id = lax.axis_index("s")
    wid = s_id * NC + c_id
    base = wid * EW
    pltpu.sync_copy(sums_hbm, s_all)
    pltpu.sync_copy(p_hbm.at[pl.ds(base, EW)], p_v)
    pltpu.sync_copy(dst_hbm.at[pl.ds(base, EW)], dst_v)

    @pl.loop(0, N // 16)
    def _(i):
        o = i * 16
        tot = s_all[pl.ds(o, 16)] + s_all[pl.ds(N + o, 16)]
        iv = 1.0 / (tot + 1e-9)
        inv_t[pl.ds(o, 16)] = iv
        r_t[pl.ds(o, 16)] = tot * iv

    @pl.when(wid == 0)
    def _():
        pltpu.sync_copy(r_t, r_hbm)

    @pl.loop(0, EW // 16, unroll=4)
    def _(g):
        o = g * 16
        idx = dst_v[pl.ds(o, 16)]
        iv = plsc.load_gather(inv_t, [idx])
        w_v[pl.ds(o, 16)] = p_v[pl.ds(o, 16)] * iv

    pltpu.sync_copy(w_v, w_hbm.at[pl.ds(base, EW)])


def kernel(node_features, edge_index, edge_weight, Wq, bq, Wk, bk, Wv, bv, cbw, Wo, bo):
    f32 = jnp.float32
    pad_idx = jnp.arange(EPAD - E, dtype=jnp.int32) % N
    src = jnp.concatenate([edge_index[0], pad_idx])
    dst = jnp.concatenate([edge_index[1], pad_idx])
    ew = jnp.concatenate([edge_weight.astype(f32), jnp.zeros((EPAD - E,), f32)])
    cbw16 = jnp.broadcast_to(cbw.astype(f32), (16,))
    bq2 = bq.reshape(1, D)
    bk2 = bk.reshape(1, D)
    bv2 = bv.reshape(1, D)
    bo2 = bo.reshape(1, D)

    q, k = pl.pallas_call(
        _qk_body,
        grid=(NB,),
        in_specs=[
            pl.BlockSpec((RB, D), lambda i: (i, 0)),
            pl.BlockSpec((D, D), lambda i: (0, 0)),
            pl.BlockSpec((D, D), lambda i: (0, 0)),
            pl.BlockSpec((1, D), lambda i: (0, 0)),
            pl.BlockSpec((1, D), lambda i: (0, 0)),
        ],
        out_specs=[pl.BlockSpec((RB, DP), lambda i: (i, 0))] * 2,
        out_shape=[jax.ShapeDtypeStruct((N, DP), f32)] * 2,
    )(node_features, Wq, Wk, bq2, bk2)

    mesh = plsc.VectorSubcoreMesh(core_axis_name="c", subcore_axis_name="s")
    p, sums = pl.kernel(
        _edge_body,
        out_type=[
            jax.ShapeDtypeStruct((EPAD,), f32),
            jax.ShapeDtypeStruct((NC, N), f32),
        ],
        mesh=mesh,
        scratch_types=[
            pltpu.VMEM((EW,), jnp.int32),
            pltpu.VMEM((EW,), jnp.int32),
            pltpu.VMEM((EW,), f32),
            [pltpu.VMEM((C,), jnp.int32)] * 2,
            pltpu.VMEM((C,), f32),
            [pltpu.VMEM((C, DP), f32)] * 2,
            [pltpu.VMEM((C, DP), f32)] * 2,
            pltpu.VMEM((16,), f32),
            pltpu.VMEM((2000,), f32),
            pltpu.VMEM((17 * 16,), f32),
            pltpu.VMEM_SHARED((N,), f32),
            [pltpu.SemaphoreType.DMA] * 2,
        ],
        compiler_params=pltpu.CompilerParams(
            use_tc_tiling_on_sc=False, needs_layout_passes=False),
    )(q, k, src, dst, ew, cbw16)

    mesh_b = plsc.VectorSubcoreMesh(core_axis_name="c", subcore_axis_name="s")
    w, r = pl.kernel(
        _wts_body,
        out_type=[
            jax.ShapeDtypeStruct((EPAD,), f32),
            jax.ShapeDtypeStruct((N,), f32),
        ],
        mesh=mesh_b,
        scratch_types=[
            pltpu.VMEM((NC * N,), f32),
            pltpu.VMEM((N,), f32),
            pltpu.VMEM((N,), f32),
            pltpu.VMEM((EW,), f32),
            pltpu.VMEM((EW,), jnp.int32),
            pltpu.VMEM((EW,), f32),
        ],
        compiler_params=pltpu.CompilerParams(
            use_tc_tiling_on_sc=False, needs_layout_passes=False),
    )(p, dst, sums.reshape(-1))

    out = pl.pallas_call(
        _out_body,
        grid=(NB,),
        in_specs=[
            pl.BlockSpec((RB, D), lambda i: (i, 0)),
            pl.BlockSpec((RB, 1), lambda i: (i, 0)),
            pl.BlockSpec((D, D), lambda i: (0, 0)),
            pl.BlockSpec((1, D), lambda i: (0, 0)),
            pl.BlockSpec((D, D), lambda i: (0, 0)),
            pl.BlockSpec((1, D), lambda i: (0, 0)),
        ],
        out_specs=pl.BlockSpec((RB, D), lambda i: (i, 0)),
        out_shape=jax.ShapeDtypeStruct((N, D), f32),
    )(node_features, r.reshape(N, 1), Wv, bv2, Wo, bo2)

    return out, w[:E]
